# Initial kernel scaffold; baseline (speedup 1.0000x reference)
import jax
import jax.numpy as jnp
from jax.experimental import pallas as pl


def kernel(x, modality_ids, W1, W2, W3):
    raise NotImplementedError("write your pallas kernel here")


# speedup vs baseline: 1.0147x; 1.0147x over previous
import jax
import jax.numpy as jnp
from jax.experimental import pallas as pl

_DIM = 768
_HIDDEN = 2048
_BLK = 512


def _ffn_block(xb, w1, w3, w2):
    x1 = jax.lax.dot_general(xb, w1, (((1,), (1,)), ((), ())),
                             preferred_element_type=jnp.float32)
    x3 = jax.lax.dot_general(xb, w3, (((1,), (1,)), ((), ())),
                             preferred_element_type=jnp.float32)
    h = (x1 * jax.lax.logistic(x1) * x3).astype(jnp.bfloat16)
    return jax.lax.dot_general(h, w2, (((1,), (1,)), ((), ())),
                               preferred_element_type=jnp.float32)


def _masked_ffn_kernel(ids_ref, x_ref, w1_ref, w3_ref, w2_ref, o_ref):
    xb = x_ref[...].astype(jnp.bfloat16)
    out0 = _ffn_block(xb, w1_ref[0], w3_ref[0], w2_ref[0])
    out1 = _ffn_block(xb, w1_ref[1], w3_ref[1], w2_ref[1])
    mask = ids_ref[...] == 1.0
    o_ref[...] = jnp.where(mask, out1, out0)


def kernel(x, modality_ids, W1, W2, W3):
    bsz, seq_len, dim = x.shape
    n_tok = bsz * seq_len
    n_blk = n_tok // _BLK
    xf = x.reshape(n_tok, dim)
    ids = modality_ids.astype(jnp.float32).reshape(n_tok, 1)
    w1 = W1.astype(jnp.bfloat16)
    w3 = W3.astype(jnp.bfloat16)
    w2 = W2.astype(jnp.bfloat16)

    out = pl.pallas_call(
        _masked_ffn_kernel,
        grid=(n_blk,),
        in_specs=[
            pl.BlockSpec((_BLK, 1), lambda i: (i, 0)),
            pl.BlockSpec((_BLK, dim), lambda i: (i, 0)),
            pl.BlockSpec((2, _HIDDEN, dim), lambda i: (0, 0, 0)),
            pl.BlockSpec((2, _HIDDEN, dim), lambda i: (0, 0, 0)),
            pl.BlockSpec((2, dim, _HIDDEN), lambda i: (0, 0, 0)),
        ],
        out_specs=pl.BlockSpec((_BLK, dim), lambda i: (i, 0)),
        out_shape=jax.ShapeDtypeStruct((n_tok, dim), jnp.float32),
    )(ids, xf, w1, w3, w2)
    return out.reshape(bsz, seq_len, dim)
